# baseline (speedup 1.0000x reference)
import functools

import jax
import jax.numpy as jnp
from jax import lax
from jax.experimental import pallas as pl
from jax.experimental.pallas import tpu as pltpu
import jax.experimental.pallas.tpu_sc as plsc

N = 10000
E = 160000
D = 256
H = 128
NC = 2
NS = 16
CH = 192
CHN = -(-E // (NS * CH))
EP = NS * CH * CHN
NZB = 79
N_ACC = NZB * 128
RB = 1000


def _tc_linear(x, wt, b2):
    def body(x_ref, wt_ref, b_ref, out_ref):
        out_ref[...] = (
            jnp.dot(x_ref[...], wt_ref[...], preferred_element_type=jnp.float32)
            + b_ref[...]
        )

    return pl.pallas_call(
        body,
        grid=(NC, N // RB),
        in_specs=[
            pl.BlockSpec((RB, D), lambda h, i: (i, 0)),
            pl.BlockSpec((D, H), lambda h, i: (0, h)),
            pl.BlockSpec((1, H), lambda h, i: (0, h)),
        ],
        out_specs=pl.BlockSpec((RB, H), lambda h, i: (h * (N // RB) + i, 0)),
        out_shape=jax.ShapeDtypeStruct((2 * N, H), jnp.float32),
    )(x, wt, b2)


def _make_sc_scatter():
    mesh = plsc.VectorSubcoreMesh(core_axis_name="c", subcore_axis_name="s")

    @functools.partial(
        pl.kernel,
        out_type=jax.ShapeDtypeStruct((N, D), jnp.float32),
        mesh=mesh,
        scratch_types=[
            pltpu.VMEM((CHN * CH,), jnp.int32),
            pltpu.VMEM((CHN * CH,), jnp.int32),
            pltpu.VMEM((CH, H), jnp.float32),
            pltpu.VMEM_SHARED((N_ACC, H), jnp.float32),
            pltpu.SemaphoreType.DMA,
        ],
    )
    def sc_scatter(table, cols, rows, zblk, out, cidx, ridx, gbuf, acc, sem):
        c = lax.axis_index("c")
        s = lax.axis_index("s")
        pltpu.sync_copy(cols.at[c, s], cidx)
        pltpu.sync_copy(rows.at[s], ridx)
        for t in range(5):
            blk = s * 5 + t

            @pl.when(blk < NZB)
            def _():
                pltpu.sync_copy(zblk, acc.at[pl.ds(blk * 128, 128)])
        plsc.subcore_barrier()

        @pl.loop(0, CHN)
        def _(j):
            pltpu.async_copy(
                table.at[cidx.at[pl.ds(j * CH, CH)]], gbuf, sem
            ).wait()
            pltpu.sync_copy(
                gbuf.at[pl.ds(0, 128)],
                acc.at[ridx.at[pl.ds(j * CH, 128)]],
                add=True,
            )
            pltpu.sync_copy(
                gbuf.at[pl.ds(128, CH - 128)],
                acc.at[ridx.at[pl.ds(j * CH + 128, CH - 128)]],
                add=True,
            )

        plsc.subcore_barrier()
        base = s * 640

        @pl.when(base + 640 <= N)
        def _():
            pltpu.sync_copy(
                acc.at[pl.ds(base, 640)],
                out.at[pl.ds(base, 640), pl.ds(c * H, H)],
            )

        @pl.when(base + 640 > N)
        def _():
            pltpu.sync_copy(
                acc.at[pl.ds(base, N - 640 * (NS - 1))],
                out.at[pl.ds(base, N - 640 * (NS - 1)), pl.ds(c * H, H)],
            )

    return sc_scatter


_sc_scatter = _make_sc_scatter()


@jax.jit
def kernel(x, edge_index, batch, W, b):
    row = edge_index[0]
    col = edge_index[1]
    pad = EP - E
    row_p = jnp.concatenate([row, jnp.full((pad,), N, jnp.int32)])
    col_p = jnp.concatenate([col, jnp.zeros((pad,), jnp.int32)])
    rows_arr = row_p.reshape(NS, CHN * CH)
    cols_arr = jnp.stack([col_p, col_p + N]).reshape(NC, NS, CHN * CH)
    zblk = jnp.zeros((128, H), jnp.float32)

    table = _tc_linear(x, W.T, b.reshape(1, D))
    return _sc_scatter(table, cols_arr, rows_arr, zblk)


# speedup vs baseline: 1.0084x; 1.0084x over previous
import functools

import jax
import jax.numpy as jnp
from jax import lax
from jax.experimental import pallas as pl
from jax.experimental.pallas import tpu as pltpu
import jax.experimental.pallas.tpu_sc as plsc

N = 10000
E = 160000
D = 256
H = 128
NC = 2
NS = 16
CH = 192
CHN = -(-E // (NS * CH))
EP = NS * CH * CHN
NZB = 79
N_ACC = NZB * 128
RB = 1000


def _tc_linear(x, wt, b2):
    def body(x_ref, wt_ref, b_ref, out_ref):
        out_ref[...] = (
            jnp.dot(x_ref[...], wt_ref[...], preferred_element_type=jnp.float32)
            + b_ref[...]
        )

    return pl.pallas_call(
        body,
        grid=(NC, N // RB),
        in_specs=[
            pl.BlockSpec((RB, D), lambda h, i: (i, 0)),
            pl.BlockSpec((D, H), lambda h, i: (0, h)),
            pl.BlockSpec((1, H), lambda h, i: (0, h)),
        ],
        out_specs=pl.BlockSpec((RB, H), lambda h, i: (h * (N // RB) + i, 0)),
        out_shape=jax.ShapeDtypeStruct((2 * N, H), jnp.float32),
    )(x, wt, b2)


def _make_sc_scatter():
    mesh = plsc.VectorSubcoreMesh(core_axis_name="c", subcore_axis_name="s")

    @functools.partial(
        pl.kernel,
        out_type=jax.ShapeDtypeStruct((N, D), jnp.float32),
        mesh=mesh,
        scratch_types=[
            pltpu.VMEM((CHN * CH,), jnp.int32),
            pltpu.VMEM((CHN * CH,), jnp.int32),
            pltpu.VMEM((CH, H), jnp.float32),
            pltpu.VMEM_SHARED((N_ACC, H), jnp.float32),
            pltpu.SemaphoreType.DMA,
        ],
    )
    def sc_scatter(table, cols, rows, zblk, out, cidx, ridx, gbuf, acc, sem):
        c = lax.axis_index("c")
        s = lax.axis_index("s")
        pltpu.sync_copy(cols.at[c, s], cidx)
        pltpu.sync_copy(rows.at[s], ridx)
        for t in range(5):
            blk = s * 5 + t

            @pl.when(blk < NZB)
            def _():
                pltpu.sync_copy(zblk, acc.at[pl.ds(blk * 128, 128)])
        plsc.subcore_barrier()

        @pl.loop(0, CHN)
        def _(j):
            pltpu.async_copy(
                table.at[cidx.at[pl.ds(j * CH, CH)]], gbuf, sem
            ).wait()
            pltpu.sync_copy(
                gbuf,
                acc.at[ridx.at[pl.ds(j * CH, CH)]],
                add=True,
            )

        plsc.subcore_barrier()
        base = s * 640

        @pl.when(base + 640 <= N)
        def _():
            pltpu.sync_copy(
                acc.at[pl.ds(base, 640)],
                out.at[pl.ds(base, 640), pl.ds(c * H, H)],
            )

        @pl.when(base + 640 > N)
        def _():
            pltpu.sync_copy(
                acc.at[pl.ds(base, N - 640 * (NS - 1))],
                out.at[pl.ds(base, N - 640 * (NS - 1)), pl.ds(c * H, H)],
            )

    return sc_scatter


_sc_scatter = _make_sc_scatter()


@jax.jit
def kernel(x, edge_index, batch, W, b):
    row = edge_index[0]
    col = edge_index[1]
    pad = EP - E
    row_p = jnp.concatenate([row, jnp.full((pad,), N, jnp.int32)])
    col_p = jnp.concatenate([col, jnp.zeros((pad,), jnp.int32)])
    rows_arr = row_p.reshape(NS, CHN * CH)
    cols_arr = jnp.stack([col_p, col_p + N]).reshape(NC, NS, CHN * CH)
    zblk = jnp.zeros((128, H), jnp.float32)

    table = _tc_linear(x, W.T, b.reshape(1, D))
    return _sc_scatter(table, cols_arr, rows_arr, zblk)


# speedup vs baseline: 1.1159x; 1.1067x over previous
import functools

import jax
import jax.numpy as jnp
from jax import lax
from jax.experimental import pallas as pl
from jax.experimental.pallas import tpu as pltpu
import jax.experimental.pallas.tpu_sc as plsc

N = 10000
E = 160000
D = 256
H = 128
NC = 2
NS = 16
CH = 128
CHN = -(-E // (NS * CH))
EP = NS * CH * CHN
N_ACC = NS * CH * 5
RB = 1000


def _tc_linear(x, wt, b2):
    def body(x_ref, wt_ref, b_ref, out_ref):
        out_ref[...] = (
            jnp.dot(x_ref[...], wt_ref[...], preferred_element_type=jnp.float32)
            + b_ref[...]
        )

    return pl.pallas_call(
        body,
        grid=(NC, N // RB),
        in_specs=[
            pl.BlockSpec((RB, D), lambda h, i: (i, 0)),
            pl.BlockSpec((D, H), lambda h, i: (0, h)),
            pl.BlockSpec((1, H), lambda h, i: (0, h)),
        ],
        out_specs=pl.BlockSpec((RB, H), lambda h, i: (h * (N // RB) + i, 0)),
        out_shape=jax.ShapeDtypeStruct((2 * N, H), jnp.float32),
    )(x, wt, b2)


def _make_sc_scatter():
    mesh = plsc.VectorSubcoreMesh(core_axis_name="c", subcore_axis_name="s")

    @functools.partial(
        pl.kernel,
        out_type=jax.ShapeDtypeStruct((N, D), jnp.float32),
        mesh=mesh,
        scratch_types=[
            pltpu.VMEM((CHN, CH), jnp.int32),
            pltpu.VMEM((CHN, CH), jnp.int32),
            pltpu.VMEM((CH, H), jnp.float32),
            pltpu.VMEM_SHARED((N_ACC, H), jnp.float32),
            pltpu.SemaphoreType.DMA,
        ],
    )
    def sc_scatter(table, cols, rows, zblk, out, cidx, ridx, gbuf, acc, sem):
        c = lax.axis_index("c")
        s = lax.axis_index("s")
        pltpu.sync_copy(cols.at[c, s], cidx)
        pltpu.sync_copy(rows.at[s], ridx)
        for t in range(N_ACC // (NS * CH)):
            pltpu.sync_copy(zblk, acc.at[pl.ds((s * 5 + t) * CH, CH)])
        plsc.subcore_barrier()

        @pl.loop(0, CHN)
        def _(j):
            pltpu.async_copy(table.at[cidx.at[j]], gbuf, sem).wait()
            pltpu.sync_copy(gbuf, acc.at[ridx.at[j]], add=True)

        plsc.subcore_barrier()
        base = s * 640

        @pl.when(base + 640 <= N)
        def _():
            pltpu.sync_copy(
                acc.at[pl.ds(base, 640)],
                out.at[pl.ds(base, 640), pl.ds(c * H, H)],
            )

        @pl.when(base + 640 > N)
        def _():
            pltpu.sync_copy(
                acc.at[pl.ds(base, N - 640 * (NS - 1))],
                out.at[pl.ds(base, N - 640 * (NS - 1)), pl.ds(c * H, H)],
            )

    return sc_scatter


_sc_scatter = _make_sc_scatter()


@jax.jit
def kernel(x, edge_index, batch, W, b):
    row = edge_index[0]
    col = edge_index[1]
    pad = EP - E
    row_p = jnp.concatenate([row, jnp.full((pad,), N, jnp.int32)])
    col_p = jnp.concatenate([col, jnp.zeros((pad,), jnp.int32)])
    rows_arr = row_p.reshape(NS, CHN, CH)
    cols_arr = jnp.stack([col_p, col_p + N]).reshape(NC, NS, CHN, CH)
    zblk = jnp.zeros((CH, H), jnp.float32)

    table = _tc_linear(x, W.T, b.reshape(1, D))
    return _sc_scatter(table, cols_arr, rows_arr, zblk)
